# baseline (speedup 1.0000x reference)
import functools

import jax
import jax.numpy as jnp
from jax import lax
from jax.experimental import pallas as pl
from jax.experimental.pallas import tpu as pltpu
from jax.experimental.pallas import tpu_sc as plsc

N = 2048
NC = 2
NS = 16
NW = NC * NS
L = 16
CHUNK = N // NW
KVECS = N // L
LN2 = 0.6931471805599453


def _sum_all(x, iota):
    for sh in (8, 4, 2, 1):
        x = x + jnp.take(x, iota ^ sh)
    return x


def _max_all(x, iota):
    for sh in (8, 4, 2, 1):
        x = jnp.maximum(x, jnp.take(x, iota ^ sh))
    return x


def _vlog(x):
    bits = plsc.bitcast(x, jnp.int32)
    ex = (bits >> 23) - 127
    mant = plsc.bitcast((bits & 0x007FFFFF) | 0x3F800000, jnp.float32)
    big = mant > 1.4142135623730951
    mant = jnp.where(big, mant * 0.5, mant)
    ex_f = ex.astype(jnp.float32) + jnp.where(big, 1.0, 0.0)
    z = (mant - 1.0) / (mant + 1.0)
    z2 = z * z
    p = z * (2.0 + z2 * (0.66666667 + z2 * (0.4 + z2 * (0.28571429 + z2 * 0.22222222))))
    return ex_f * LN2 + p


def _body(scores_hbm, labels_hbm, out_hbm, sv, lv, ev, outv):
    wid = lax.axis_index("s") * NC + lax.axis_index("c")
    pltpu.sync_copy(scores_hbm, sv)
    pltpu.sync_copy(labels_hbm, lv)

    iota = lax.iota(jnp.int32, L)

    def max_step(kb, acc):
        return jnp.maximum(acc, sv[pl.ds(kb * L, L)])
    mv = _max_all(lax.fori_loop(0, KVECS, max_step, jnp.full((L,), -3.0e38, jnp.float32)), iota)

    def exp_step(kb, _):
        ev[pl.ds(kb * L, L)] = jnp.exp(sv[pl.ds(kb * L, L)] - mv)
        return 0
    lax.fori_loop(0, KVECS, exp_step, 0)

    base = wid * CHUNK

    tacc = jnp.zeros((L,), jnp.float32)
    for g in range(CHUNK // L):
        gbase = base + g * L
        l_grp = lv[pl.ds(gbase, L)]

        def lane_step(jj, wacc, gbase=gbase, l_grp=l_grp):
            i = gbase + jj
            l_i = jnp.take(l_grp, jnp.full((L,), jj, jnp.int32))

            def k_step(kb, acc):
                off = kb * L
                lvec = lv[pl.ds(off, L)]
                evec = ev[pl.ds(off, L)]
                kvec = iota + off
                msk = (lvec < l_i) | ((lvec == l_i) & (kvec >= i))
                return acc + jnp.where(msk, evec, 0.0)

            acc = lax.fori_loop(0, KVECS, k_step, jnp.zeros((L,), jnp.float32))
            return jnp.where(iota == jj, _sum_all(acc, iota), wacc)

        wvec = lax.fori_loop(0, L, lane_step, jnp.zeros((L,), jnp.float32))
        svec = sv[pl.ds(gbase, L)]
        tacc = tacc + (_vlog(wvec) + mv - svec)

    outv[...] = _sum_all(tacc, iota) * (1.0 / N)
    pltpu.sync_copy(outv, out_hbm.at[wid])


@functools.partial(
    pl.kernel,
    out_type=jax.ShapeDtypeStruct((NW, L), jnp.float32),
    mesh=plsc.VectorSubcoreMesh(
        core_axis_name="c", subcore_axis_name="s", num_cores=NC, num_subcores=NS
    ),
    compiler_params=pltpu.CompilerParams(needs_layout_passes=False),
    scratch_types=[
        pltpu.VMEM((N,), jnp.float32),
        pltpu.VMEM((N,), jnp.float32),
        pltpu.VMEM((N,), jnp.float32),
        pltpu.VMEM((L,), jnp.float32),
    ],
)
def _listmle_sc(scores_hbm, labels_hbm, out_hbm, sv, lv, ev, outv):
    _body(scores_hbm, labels_hbm, out_hbm, sv, lv, ev, outv)


def kernel(scores, labels):
    partials = _listmle_sc(scores, labels)
    return jnp.sum(partials[:, 0])


# speedup vs baseline: 1.3908x; 1.3908x over previous
import functools

import jax
import jax.numpy as jnp
from jax import lax
from jax.experimental import pallas as pl
from jax.experimental.pallas import tpu as pltpu
from jax.experimental.pallas import tpu_sc as plsc

N = 2048
NC = 2
NS = 16
NW = NC * NS
L = 16
CHUNK = N // NW
KVECS = N // L
LN2 = 0.6931471805599453


def _sum_all(x, iota):
    for sh in (8, 4, 2, 1):
        x = x + jnp.take(x, iota ^ sh)
    return x


def _max_all(x, iota):
    for sh in (8, 4, 2, 1):
        x = jnp.maximum(x, jnp.take(x, iota ^ sh))
    return x


def _vlog(x):
    bits = plsc.bitcast(x, jnp.int32)
    ex = (bits >> 23) - 127
    mant = plsc.bitcast((bits & 0x007FFFFF) | 0x3F800000, jnp.float32)
    big = mant > 1.4142135623730951
    mant = jnp.where(big, mant * 0.5, mant)
    ex_f = ex.astype(jnp.float32) + jnp.where(big, 1.0, 0.0)
    z = (mant - 1.0) / (mant + 1.0)
    z2 = z * z
    p = z * (2.0 + z2 * (0.66666667 + z2 * (0.4 + z2 * (0.28571429 + z2 * 0.22222222))))
    return ex_f * LN2 + p


def _body(scores_hbm, labels_hbm, out_hbm, sv, lv, ev, outv):
    wid = lax.axis_index("s") * NC + lax.axis_index("c")
    pltpu.sync_copy(scores_hbm, sv)
    pltpu.sync_copy(labels_hbm, lv)

    iota = lax.iota(jnp.int32, L)

    def max_step(kb, acc):
        return jnp.maximum(acc, sv[pl.ds(kb * L, L)])
    mv = _max_all(lax.fori_loop(0, KVECS, max_step, jnp.full((L,), -3.0e38, jnp.float32)), iota)

    def exp_step(kb, _):
        ev[pl.ds(kb * L, L)] = jnp.exp(sv[pl.ds(kb * L, L)] - mv)
        return 0
    lax.fori_loop(0, KVECS, exp_step, 0)

    base = wid * CHUNK

    tacc = jnp.zeros((L,), jnp.float32)
    for g in range(CHUNK // L):
        gbase = base + g * L
        l_grp = lv[pl.ds(gbase, L)]

        def lane_step(jj, wacc, gbase=gbase, l_grp=l_grp):
            i = gbase + jj
            l_i = jnp.take(l_grp, jnp.full((L,), jj, jnp.int32))

            def k_step(kb, acc):
                off = kb * L
                lvec = lv[pl.ds(off, L)]
                evec = ev[pl.ds(off, L)]
                kvec = iota + off
                msk = (lvec < l_i) | ((lvec == l_i) & (kvec >= i))
                return acc + jnp.where(msk, evec, 0.0)

            acc = lax.fori_loop(0, KVECS, k_step, jnp.zeros((L,), jnp.float32), unroll=8)
            return jnp.where(iota == jj, _sum_all(acc, iota), wacc)

        wvec = lax.fori_loop(0, L, lane_step, jnp.zeros((L,), jnp.float32))
        svec = sv[pl.ds(gbase, L)]
        tacc = tacc + (_vlog(wvec) + mv - svec)

    outv[...] = _sum_all(tacc, iota) * (1.0 / N)
    pltpu.sync_copy(outv, out_hbm.at[wid])


@functools.partial(
    pl.kernel,
    out_type=jax.ShapeDtypeStruct((NW, L), jnp.float32),
    mesh=plsc.VectorSubcoreMesh(
        core_axis_name="c", subcore_axis_name="s", num_cores=NC, num_subcores=NS
    ),
    compiler_params=pltpu.CompilerParams(needs_layout_passes=False),
    scratch_types=[
        pltpu.VMEM((N,), jnp.float32),
        pltpu.VMEM((N,), jnp.float32),
        pltpu.VMEM((N,), jnp.float32),
        pltpu.VMEM((L,), jnp.float32),
    ],
)
def _listmle_sc(scores_hbm, labels_hbm, out_hbm, sv, lv, ev, outv):
    _body(scores_hbm, labels_hbm, out_hbm, sv, lv, ev, outv)


def kernel(scores, labels):
    partials = _listmle_sc(scores, labels)
    return jnp.sum(partials[:, 0])
